# Initial kernel scaffold; baseline (speedup 1.0000x reference)
import jax
import jax.numpy as jnp
from jax.experimental import pallas as pl


def kernel(x, Wg, W1, W2):
    raise NotImplementedError("write your pallas kernel here")


# speedup vs baseline: 1.2063x; 1.2063x over previous
import functools

import jax
import jax.numpy as jnp
from jax.experimental import pallas as pl
from jax.experimental.pallas import tpu as pltpu

NUM_EXPERTS = 8
TOP_K = 2
D_MODEL = 1024
D_FF = 2048
SEQ = 2048

F_BLK = 512
NF = D_FF // F_BLK


def _routing_body(x_ref, wg_ref, gate_ref):
    scores = jnp.dot(x_ref[...], wg_ref[...],
                     preferred_element_type=jnp.float32)
    lane = jax.lax.broadcasted_iota(jnp.int32, scores.shape, 1)
    m1 = jnp.max(scores, axis=-1, keepdims=True)
    a1 = jnp.min(jnp.where(scores == m1, lane, NUM_EXPERTS), axis=-1,
                 keepdims=True)
    oh1 = (lane == a1).astype(jnp.float32)
    s2 = jnp.where(lane == a1, -jnp.inf, scores)
    m2 = jnp.max(s2, axis=-1, keepdims=True)
    a2 = jnp.min(jnp.where(s2 == m2, lane, NUM_EXPERTS), axis=-1,
                 keepdims=True)
    oh2 = (lane == a2).astype(jnp.float32)
    e2 = jnp.exp(m2 - m1)
    w1 = 1.0 / (1.0 + e2)
    w2 = 1.0 - w1
    gate_ref[...] = oh1 * w1 + oh2 * w2


def _moe_dense_body(gate_ref, x_ref, w1_ref, w2_ref, o_ref):
    e = pl.program_id(0)
    f = pl.program_id(1)

    @pl.when((e == 0) & (f == 0))
    def _():
        o_ref[...] = jnp.zeros_like(o_ref)

    h = jnp.dot(x_ref[...], w1_ref[0], preferred_element_type=jnp.float32)
    h = h * jax.nn.sigmoid(h)
    p = jnp.dot(h, w2_ref[0], preferred_element_type=jnp.float32)
    lane = jax.lax.broadcasted_iota(jnp.int32, (NUM_EXPERTS, 1), 0)
    sel = (lane == e).astype(jnp.float32)
    col = jnp.dot(gate_ref[...], sel, preferred_element_type=jnp.float32)
    o_ref[...] += col * p


@jax.jit
def kernel(x, Wg, W1, W2):
    orig_shape = x.shape
    xf = x.reshape(-1, x.shape[-1])
    T = xf.shape[0]

    gate = pl.pallas_call(
        _routing_body,
        out_shape=jax.ShapeDtypeStruct((T, NUM_EXPERTS), jnp.float32),
    )(xf, Wg)

    out = pl.pallas_call(
        _moe_dense_body,
        grid=(NUM_EXPERTS, NF),
        in_specs=[
            pl.BlockSpec((T, NUM_EXPERTS), lambda e, f: (0, 0)),
            pl.BlockSpec((T, D_MODEL), lambda e, f: (0, 0)),
            pl.BlockSpec((1, D_MODEL, F_BLK), lambda e, f: (e, 0, f)),
            pl.BlockSpec((1, F_BLK, D_MODEL), lambda e, f: (e, f, 0)),
        ],
        out_specs=pl.BlockSpec((T, D_MODEL), lambda e, f: (0, 0)),
        out_shape=jax.ShapeDtypeStruct((T, D_MODEL), jnp.float32),
    )(gate, xf, W1, W2)

    return out.reshape(orig_shape)
